# baseline (speedup 1.0000x reference)
import functools

import jax
import jax.numpy as jnp
from jax import lax
from jax.experimental import pallas as pl
from jax.experimental.pallas import tpu as pltpu
from jax.experimental.pallas import tpu_sc as plsc

D = 32
L = 16
TW = 128


def _mesh_info():
    info = plsc.get_sparse_core_info()
    return info.num_cores, info.num_subcores


def _fetch_sorted_blocks(tab_h, blk, sem, cs, lane):
    zero = jnp.zeros((), jnp.int32)
    slots = jnp.zeros((L,), jnp.int32)
    slot = zero
    nfetch = zero
    for j in range(L):
        if j == 0:
            is_new = jnp.bool_(True)
        else:
            is_new = cs[j] != cs[j - 1]
        slot = jnp.where(is_new, nfetch, slot)
        nfetch = nfetch + jnp.where(is_new, 1, 0)
        off = pl.multiple_of(cs[j], TW)

        @pl.when(is_new)
        def _(off=off, slot=slot):
            pltpu.async_copy(tab_h.at[:, pl.ds(off, TW)], blk.at[slot], sem)

        slots = jnp.where(lane == j, slot, slots)

    def wait_one(k, carry):
        pltpu.make_async_copy(tab_h.at[:, pl.ds(0, TW)], blk.at[0], sem).wait()
        return carry

    lax.fori_loop(0, nfetch, wait_one, 0)
    return slots


@functools.lru_cache(maxsize=None)
def _build_item(B):
    NC, NS = _mesh_info()
    NW = NC * NS
    bpw = B // NW
    NG = bpw // L

    mesh = plsc.VectorSubcoreMesh(core_axis_name="c", subcore_axis_name="s")

    @functools.partial(
        pl.kernel,
        mesh=mesh,
        out_type=jax.ShapeDtypeStruct((B, TW), jnp.float32),
        compiler_params=pltpu.CompilerParams(
            needs_layout_passes=False, disable_bounds_checks=True),
        scratch_types=[
            pltpu.VMEM((bpw,), jnp.int32),
            pltpu.VMEM((NG, L), jnp.int32),
            pltpu.VMEM((L, D, TW), jnp.float32),
            pltpu.VMEM((L, TW), jnp.float32),
            pltpu.SemaphoreType.DMA,
        ],
    )
    def item_kernel(sidx_h, rows_h, itabT_h, strip_h,
                    sixv, rowv, blk, stg, sem):
        wid = lax.axis_index("s") * NC + lax.axis_index("c")
        base = wid * bpw

        pltpu.sync_copy(sidx_h.at[pl.ds(base, bpw)], sixv)
        pltpu.sync_copy(rows_h.at[wid], rowv)
        lane = lax.iota(jnp.int32, L)

        def group(g, carry):
            svec = sixv[pl.ds(g * L, L)]
            slots = _fetch_sorted_blocks(itabT_h, blk, sem, svec & -TW, lane)
            sph = svec & (TW - 1)
            rvec = rowv[g, :]
            for d in range(D):
                dv = jnp.full((L,), d, dtype=jnp.int32)
                vals = plsc.load_gather(blk, [slots, dv, sph])
                plsc.store_scatter(stg, [lane, (dv + rvec) & (TW - 1)], vals)
            pltpu.async_copy(stg, strip_h.at[rowv.at[g]], sem).wait()
            return carry

        lax.fori_loop(0, NG, group, 0)

    return item_kernel


@functools.lru_cache(maxsize=None)
def _build_user(B):
    NC, NS = _mesh_info()
    NW = NC * NS
    bpw = B // NW
    NG = bpw // L

    mesh = plsc.VectorSubcoreMesh(core_axis_name="c", subcore_axis_name="s")

    @functools.partial(
        pl.kernel,
        mesh=mesh,
        out_type=jax.ShapeDtypeStruct((B,), jnp.float32),
        compiler_params=pltpu.CompilerParams(
            needs_layout_passes=False, disable_bounds_checks=True),
        scratch_types=[
            pltpu.VMEM((bpw,), jnp.int32),
            pltpu.VMEM((bpw,), jnp.int32),
            pltpu.VMEM((L, D, TW), jnp.float32),
            pltpu.VMEM((D, L), jnp.float32),
            pltpu.VMEM((L, TW), jnp.float32),
            pltpu.VMEM((D,), jnp.float32),
            pltpu.VMEM((L,), jnp.float32),
            pltpu.VMEM((bpw,), jnp.float32),
            pltpu.SemaphoreType.DMA,
        ],
    )
    def user_kernel(sidx_h, rows_h, utabT_h, strip_h, w_h, b_h, out_h,
                    uixv, rowv, blk, stage, sbuf, wv, bv, outv, sem):
        wid = lax.axis_index("s") * NC + lax.axis_index("c")
        base = wid * bpw

        pltpu.sync_copy(sidx_h.at[pl.ds(base, bpw)], uixv)
        pltpu.sync_copy(rows_h.at[pl.ds(base, bpw)], rowv)
        pltpu.sync_copy(w_h, wv)
        pltpu.sync_copy(b_h, bv)

        w_lo = wv[pl.ds(0, L)]
        w_hi = wv[pl.ds(L, L)]
        bvec = bv[...]
        lane = lax.iota(jnp.int32, L)

        def group(g, carry):
            uvec = uixv[pl.ds(g * L, L)]
            slots = _fetch_sorted_blocks(utabT_h, blk, sem, uvec & -TW, lane)
            uph = uvec & (TW - 1)
            for d in range(D):
                dv = jnp.full((L,), d, dtype=jnp.int32)
                w_d = w_lo[d] if d < L else w_hi[d - L]
                stage[d, :] = plsc.load_gather(blk, [slots, dv, uph]) * w_d

            pltpu.async_copy(
                strip_h.at[rowv.at[pl.ds(g * L, L)]], sbuf, sem)
            rvec = rowv[pl.ds(g * L, L)]
            pltpu.make_async_copy(strip_h.at[pl.ds(0, L)], sbuf, sem).wait()
            acc = bvec
            for d in range(D):
                dv = jnp.full((L,), d, dtype=jnp.int32)
                i_d = plsc.load_gather(sbuf, [lane, (dv + rvec) & (TW - 1)])
                acc = acc + stage[d, :] * i_d
            outv[pl.ds(g * L, L)] = acc
            return carry

        lax.fori_loop(0, NG, group, 0)

        pltpu.sync_copy(outv, out_h.at[pl.ds(base, bpw)])

    return user_kernel


def kernel(user_idx, item_idx, user_table, item_table, W, b):
    B = user_idx.shape[0]
    NC, NS = _mesh_info()
    NW = NC * NS
    bpw = B // NW
    rows = lax.iota(jnp.int32, B)
    su, pu = lax.sort_key_val(user_idx, rows)
    si, pi = lax.sort_key_val(item_idx, rows)
    strip = _build_item(B)(si, pi.reshape(NW, bpw // L, L), item_table.T)
    out_sorted = _build_user(B)(
        su, pu, user_table.T, strip,
        W.reshape(-1), jnp.broadcast_to(b, (L,)))
    return jnp.zeros((B,), jnp.float32).at[pu].set(out_sorted)
